# baseline (speedup 1.0000x reference)
import functools
import math

import jax
import jax.numpy as jnp
from jax import lax
from jax.experimental import pallas as pl
from jax.experimental.pallas import tpu as pltpu
from jax.experimental.pallas import tpu_sc as plsc

_EMBED = 64
_G = 101
_REC = 50
_TAU = 0.8
_NC = 2
_NS = 16
_NW = _NC * _NS
_K = 56
_RPC = 4


def _sc_gather_one(ids, ridx, tab):
    B = ids.shape[0]
    rpw = B // _NW
    D = _EMBED
    K = _K
    nch = rpw // _RPC
    crows = _RPC * K
    f32 = jnp.float32
    mesh = plsc.VectorSubcoreMesh(
        core_axis_name="c", subcore_axis_name="s",
        num_cores=_NC, num_subcores=_NS)

    @functools.partial(
        pl.kernel,
        out_type=(jax.ShapeDtypeStruct((B, D), jnp.bfloat16),
                  jax.ShapeDtypeStruct((B, D), f32)),
        mesh=mesh,
        compiler_params=pltpu.CompilerParams(
            use_tc_tiling_on_sc=False, needs_layout_passes=False),
        scratch_types=[
            pltpu.VMEM((rpw,), jnp.int32),
            pltpu.VMEM((rpw, D), jnp.bfloat16),
            pltpu.VMEM((rpw * K,), jnp.int32),
            pltpu.VMEM((crows, D), jnp.bfloat16),
            pltpu.VMEM((crows, D), jnp.bfloat16),
            pltpu.VMEM((rpw, D), f32),
            pltpu.SemaphoreType.DMA,
            pltpu.SemaphoreType.DMA,
            pltpu.SemaphoreType.DMA,
        ],
    )
    def k(ids_h, ridx_h, tab_h, emb_o, rsum_o,
          idx1_v, rows1_v, idxk_v, buf0_v, buf1_v, acc_v, sem1, sem0, semb):
        wid = lax.axis_index("s") * _NC + lax.axis_index("c")
        base = wid * rpw

        pltpu.sync_copy(ids_h.at[pl.ds(base, rpw)], idx1_v)
        pltpu.sync_copy(ridx_h.at[pl.ds(base * K, rpw * K)], idxk_v)

        d1 = pltpu.async_copy(tab_h.at[idx1_v], rows1_v, sem1)

        def start(buf_v, sem, g):
            pltpu.async_copy(
                tab_h.at[idxk_v.at[pl.ds(g * crows, crows)]], buf_v, sem)

        def wait(buf_v, sem):
            pltpu.make_async_copy(
                tab_h.at[idxk_v.at[pl.ds(0, crows)]], buf_v, sem).wait()

        def accum(buf_v, g):
            def row_body(r, carry):
                accs = tuple(jnp.zeros((16,), f32) for _ in range(D // 16))

                def add8(j, a):
                    out = list(a)
                    for jj in range(8):
                        row = r * K + 8 * j + jj
                        for c in range(2):
                            ab = buf_v[row, pl.ds(32 * c, 32)]
                            lo, hi = plsc.unpack(
                                ab, format=plsc.PackFormat.INTERLEAVED)
                            out[2 * c] = out[2 * c] + lo
                            out[2 * c + 1] = out[2 * c + 1] + hi
                    return tuple(out)

                accs = lax.fori_loop(0, K // 8, add8, accs)
                for c in range(D // 16):
                    acc_v[g * _RPC + r, pl.ds(16 * c, 16)] = accs[c]
                return carry

            lax.fori_loop(0, _RPC, row_body, 0)

        start(buf0_v, sem0, 0)
        start(buf1_v, semb, 1)

        def chunk_body(t, carry):
            g0 = 2 * t
            for buf_v, sem, off in ((buf0_v, sem0, 0), (buf1_v, semb, 1)):
                g = g0 + off
                wait(buf_v, sem)
                accum(buf_v, g)

                @pl.when(g + 2 < nch)
                def _():
                    start(buf_v, sem, g + 2)

            return carry

        lax.fori_loop(0, nch // 2, chunk_body, 0)

        pltpu.sync_copy(acc_v, rsum_o.at[pl.ds(base, rpw)])
        d1.wait()
        pltpu.sync_copy(rows1_v, emb_o.at[pl.ds(base, rpw)])

    return k(ids, ridx, tab)


def _tc_dense(u1c, u2c, i1c, i2c, ue, ie, urs, irs, ur64, ir64, irow0, urow0,
              u_mac, i_mac, uV, iV, uVp, iVp, W1, b1, a1, W2, b2, a2, w3, b3):
    B = ue.shape[0]
    f32 = jnp.float32

    def body(u1c_r, u2c_r, i1c_r, i2c_r, ue_r, ie_r, urs_r, irs_r,
             ur64_r, ir64_r, irow0_r, urow0_r,
             um_r, im_r, uV_r, iV_r, uVp_r, iVp_r,
             W1_r, b1_r, a1_r, W2_r, b2_r, a2_r,
             w3_r, b3_r, out_r):
        dot = functools.partial(jnp.dot, preferred_element_type=f32)

        def w(cref):
            t = jnp.log(cref[...].astype(f32) + 1.0) * (1.0 / _TAU)
            t = t - jnp.max(t, axis=1, keepdims=True)
            e = jnp.exp(t)
            return e / jnp.sum(e, axis=1, keepdims=True)

        cnt_u = jnp.sum(jnp.where(ur64_r[...] == 0, f32(1), f32(0)),
                        axis=1, keepdims=True)
        cnt_i = jnp.sum(jnp.where(ir64_r[...] == 0, f32(1), f32(0)),
                        axis=1, keepdims=True)
        urs_c = urs_r[...] - cnt_u * irow0_r[...]
        irs_c = irs_r[...] - cnt_i * urow0_r[...]

        Vi = dot(im_r[...], iV_r[...])
        Vu = dot(um_r[...], uV_r[...])
        W1v = W1_r[...]
        h = (dot(ue_r[...].astype(f32), W1v[0:64])
             + dot(w(u1c_r), dot(Vi, W1v[64:192]))
             + dot(w(u2c_r), dot(Vu, W1v[192:320]))
             + dot(urs_c, dot(iVp_r[...], W1v[320:448]))
             + dot(ie_r[...].astype(f32), W1v[448:512])
             + dot(w(i1c_r), dot(Vu, W1v[512:640]))
             + dot(w(i2c_r), dot(Vi, W1v[640:768]))
             + dot(irs_c, dot(uVp_r[...], W1v[768:896]))
             + b1_r[...])

        def dice(hx, aref):
            n = hx.shape[0]
            mean = jnp.mean(hx, axis=0, keepdims=True)
            dv = hx - mean
            var = jnp.sum(dv * dv, axis=0, keepdims=True) / (n - 1)
            p = jax.nn.sigmoid(dv / jnp.sqrt(var) + 1e-8)
            return hx * p + aref[...] * hx * (1.0 - p)

        h = dice(h, a1_r)
        h = dice(dot(h, W2_r[...]) + b2_r[...], a2_r)
        logits = jnp.sum(h * w3_r[...], axis=1, keepdims=True) + b3_r[...]
        out_r[...] = jax.nn.sigmoid(logits)

    return pl.pallas_call(
        body, out_shape=jax.ShapeDtypeStruct((B, 1), f32),
    )(u1c, u2c, i1c, i2c, ue, ie, urs, irs, ur64, ir64, irow0, urow0,
      u_mac, i_mac, uV, iV, uVp, iVp, W1, b1, a1, W2, b2, a2, w3, b3)


def kernel(x, user_embed, item_embed, u_macro_embed, i_macro_embed,
           uQ, uK, uV, iQ, iK, iV, W1, b1, alpha1, W2, b2, alpha2, W3, b3):
    G, R = _G, _REC
    uid = x[:, 0]
    u1c = x[:, 1:1 + G]
    u2c = x[:, 1 + G:1 + 2 * G]
    ur = x[:, 1 + 2 * G:1 + 2 * G + R]
    ic = 1 + 2 * G + R
    iid = x[:, ic]
    i1c = x[:, ic + 1:ic + 1 + G]
    i2c = x[:, ic + 1 + G:ic + 1 + 2 * G]
    ir = x[:, ic + 1 + 2 * G:]
    ur64 = jnp.pad(ur, ((0, 0), (0, _K - R))).astype(jnp.int32)
    ir64 = jnp.pad(ir, ((0, 0), (0, _K - R))).astype(jnp.int32)
    bf16 = jnp.bfloat16

    ue, irs = _sc_gather_one(
        uid.astype(jnp.int32), ir64.reshape(-1), user_embed.astype(bf16))
    ie, urs = _sc_gather_one(
        iid.astype(jnp.int32), ur64.reshape(-1), item_embed.astype(bf16))
    perm = (list(range(0, 32, 2)) + list(range(1, 32, 2))
            + list(range(32, 64, 2)) + list(range(33, 64, 2)))
    perm = jnp.asarray(perm, jnp.int32)
    irow0 = item_embed[0:1].astype(bf16).astype(jnp.float32)[:, perm]
    urow0 = user_embed[0:1].astype(bf16).astype(jnp.float32)[:, perm]
    return _tc_dense(
        u1c, u2c, i1c, i2c, ue, ie, urs, irs,
        ur64, ir64, irow0, urow0,
        u_macro_embed, i_macro_embed, uV, iV,
        jnp.take(uV, perm, axis=0), jnp.take(iV, perm, axis=0),
        W1, b1.reshape(1, -1), alpha1.reshape(1, 1),
        W2, b2.reshape(1, -1), alpha2.reshape(1, 1),
        W3.reshape(1, -1), b3.reshape(1, 1))


# speedup vs baseline: 1.4528x; 1.4528x over previous
import functools
import math

import jax
import jax.numpy as jnp
from jax import lax
from jax.experimental import pallas as pl
from jax.experimental.pallas import tpu as pltpu
from jax.experimental.pallas import tpu_sc as plsc

_EMBED = 64
_G = 101
_REC = 50
_TAU = 0.8
_NC = 2
_NS = 16
_NW = _NC * _NS
_K = 56
_RPC = 4


def _sc_gather(uid, iid, ur_idx, ir_idx, user_tab, item_tab):
    B = uid.shape[0]
    rpw = B // _NW
    D = _EMBED
    K = _K
    nch = rpw // _RPC
    crows = _RPC * K
    f32 = jnp.float32
    mesh = plsc.VectorSubcoreMesh(
        core_axis_name="c", subcore_axis_name="s",
        num_cores=_NC, num_subcores=_NS)

    @functools.partial(
        pl.kernel,
        out_type=(jax.ShapeDtypeStruct((B, D), jnp.bfloat16),
                  jax.ShapeDtypeStruct((B, D), jnp.bfloat16),
                  jax.ShapeDtypeStruct((B, D), f32),
                  jax.ShapeDtypeStruct((B, D), f32)),
        mesh=mesh,
        compiler_params=pltpu.CompilerParams(
            use_tc_tiling_on_sc=False, needs_layout_passes=False),
        scratch_types=[
            pltpu.VMEM((rpw,), jnp.int32),
            pltpu.VMEM((rpw,), jnp.int32),
            pltpu.VMEM((rpw, D), jnp.bfloat16),
            pltpu.VMEM((rpw, D), jnp.bfloat16),
            pltpu.VMEM((rpw * K,), jnp.int32),
            pltpu.VMEM((rpw * K,), jnp.int32),
            pltpu.VMEM((crows, D), jnp.bfloat16),
            pltpu.VMEM((crows, D), jnp.bfloat16),
            pltpu.VMEM((crows, D), jnp.bfloat16),
            pltpu.VMEM((crows, D), jnp.bfloat16),
            pltpu.VMEM((rpw, D), f32),
            pltpu.VMEM((rpw, D), f32),
            pltpu.SemaphoreType.DMA,
            pltpu.SemaphoreType.DMA,
            pltpu.SemaphoreType.DMA,
            pltpu.SemaphoreType.DMA,
            pltpu.SemaphoreType.DMA,
            pltpu.SemaphoreType.DMA,
        ],
    )
    def k(uid_h, iid_h, ur_h, ir_h, utab_h, itab_h, ue_o, ie_o, urs_o, irs_o,
          idxu_v, idxi_v, rowsu_v, rowsi_v, idxa_v, idxb_v,
          bufa0_v, bufa1_v, bufb0_v, bufb1_v, acca_v, accb_v,
          semu, semi, sema0, sema1, semb0, semb1):
        wid = lax.axis_index("s") * _NC + lax.axis_index("c")
        base = wid * rpw

        pltpu.sync_copy(uid_h.at[pl.ds(base, rpw)], idxu_v)
        pltpu.sync_copy(iid_h.at[pl.ds(base, rpw)], idxi_v)
        pltpu.sync_copy(ur_h.at[pl.ds(base * K, rpw * K)], idxa_v)
        pltpu.sync_copy(ir_h.at[pl.ds(base * K, rpw * K)], idxb_v)

        du = pltpu.async_copy(utab_h.at[idxu_v], rowsu_v, semu)
        di = pltpu.async_copy(itab_h.at[idxi_v], rowsi_v, semi)

        def start(tab_h, idx_v, buf_v, sem, g):
            pltpu.async_copy(
                tab_h.at[idx_v.at[pl.ds(g * crows, crows)]], buf_v, sem)

        def wait(tab_h, idx_v, buf_v, sem):
            pltpu.make_async_copy(
                tab_h.at[idx_v.at[pl.ds(0, crows)]], buf_v, sem).wait()

        def accum(buf_v, acc_v, g):
            def row_body(r, carry):
                accs = tuple(jnp.zeros((16,), f32) for _ in range(D // 16))

                def add8(j, a):
                    out = list(a)
                    for jj in range(8):
                        row = r * K + 8 * j + jj
                        for c in range(2):
                            ab = buf_v[row, pl.ds(32 * c, 32)]
                            lo, hi = plsc.unpack(
                                ab, format=plsc.PackFormat.INTERLEAVED)
                            out[2 * c] = out[2 * c] + lo
                            out[2 * c + 1] = out[2 * c + 1] + hi
                    return tuple(out)

                accs = lax.fori_loop(0, K // 8, add8, accs)
                for c in range(D // 16):
                    acc_v[g * _RPC + r, pl.ds(16 * c, 16)] = accs[c]
                return carry

            lax.fori_loop(0, _RPC, row_body, 0)

        streams = (
            (itab_h, idxa_v, bufa0_v, sema0, acca_v, 0),
            (utab_h, idxb_v, bufb0_v, semb0, accb_v, 0),
            (itab_h, idxa_v, bufa1_v, sema1, acca_v, 1),
            (utab_h, idxb_v, bufb1_v, semb1, accb_v, 1),
        )
        for tab_h, idx_v, buf_v, sem, _, off in streams:
            start(tab_h, idx_v, buf_v, sem, off)

        def chunk_body(t, carry):
            g0 = 2 * t
            for tab_h, idx_v, buf_v, sem, acc_v, off in streams:
                g = g0 + off
                wait(tab_h, idx_v, buf_v, sem)
                accum(buf_v, acc_v, g)

                @pl.when(g + 2 < nch)
                def _():
                    start(tab_h, idx_v, buf_v, sem, g + 2)

            return carry

        lax.fori_loop(0, nch // 2, chunk_body, 0)

        pltpu.sync_copy(acca_v, urs_o.at[pl.ds(base, rpw)])
        pltpu.sync_copy(accb_v, irs_o.at[pl.ds(base, rpw)])
        du.wait()
        di.wait()
        pltpu.sync_copy(rowsu_v, ue_o.at[pl.ds(base, rpw)])
        pltpu.sync_copy(rowsi_v, ie_o.at[pl.ds(base, rpw)])

    return k(uid, iid, ur_idx, ir_idx, user_tab, item_tab)


def _tc_dense(u1c, u2c, i1c, i2c, ue, ie, urs, irs, ur64, ir64, irow0, urow0,
              u_mac, i_mac, uV, iV, uVp, iVp, W1, b1, a1, W2, b2, a2, w3, b3):
    B = ue.shape[0]
    f32 = jnp.float32

    def body(u1c_r, u2c_r, i1c_r, i2c_r, ue_r, ie_r, urs_r, irs_r,
             ur64_r, ir64_r, irow0_r, urow0_r,
             um_r, im_r, uV_r, iV_r, uVp_r, iVp_r,
             W1_r, b1_r, a1_r, W2_r, b2_r, a2_r,
             w3_r, b3_r, out_r):
        dot = functools.partial(jnp.dot, preferred_element_type=f32)

        def w(cref):
            t = jnp.log(cref[...].astype(f32) + 1.0) * (1.0 / _TAU)
            t = t - jnp.max(t, axis=1, keepdims=True)
            e = jnp.exp(t)
            return e / jnp.sum(e, axis=1, keepdims=True)

        cnt_u = jnp.sum(jnp.where(ur64_r[...] == 0, f32(1), f32(0)),
                        axis=1, keepdims=True)
        cnt_i = jnp.sum(jnp.where(ir64_r[...] == 0, f32(1), f32(0)),
                        axis=1, keepdims=True)
        urs_c = urs_r[...] - cnt_u * irow0_r[...]
        irs_c = irs_r[...] - cnt_i * urow0_r[...]

        Vi = dot(im_r[...], iV_r[...])
        Vu = dot(um_r[...], uV_r[...])
        W1v = W1_r[...]
        h = (dot(ue_r[...].astype(f32), W1v[0:64])
             + dot(w(u1c_r), dot(Vi, W1v[64:192]))
             + dot(w(u2c_r), dot(Vu, W1v[192:320]))
             + dot(urs_c, dot(iVp_r[...], W1v[320:448]))
             + dot(ie_r[...].astype(f32), W1v[448:512])
             + dot(w(i1c_r), dot(Vu, W1v[512:640]))
             + dot(w(i2c_r), dot(Vi, W1v[640:768]))
             + dot(irs_c, dot(uVp_r[...], W1v[768:896]))
             + b1_r[...])

        def dice(hx, aref):
            n = hx.shape[0]
            mean = jnp.mean(hx, axis=0, keepdims=True)
            dv = hx - mean
            var = jnp.sum(dv * dv, axis=0, keepdims=True) / (n - 1)
            p = jax.nn.sigmoid(dv / jnp.sqrt(var) + 1e-8)
            return hx * p + aref[...] * hx * (1.0 - p)

        h = dice(h, a1_r)
        h = dice(dot(h, W2_r[...]) + b2_r[...], a2_r)
        logits = jnp.sum(h * w3_r[...], axis=1, keepdims=True) + b3_r[...]
        out_r[...] = jax.nn.sigmoid(logits)

    return pl.pallas_call(
        body, out_shape=jax.ShapeDtypeStruct((B, 1), f32),
    )(u1c, u2c, i1c, i2c, ue, ie, urs, irs, ur64, ir64, irow0, urow0,
      u_mac, i_mac, uV, iV, uVp, iVp, W1, b1, a1, W2, b2, a2, w3, b3)


def kernel(x, user_embed, item_embed, u_macro_embed, i_macro_embed,
           uQ, uK, uV, iQ, iK, iV, W1, b1, alpha1, W2, b2, alpha2, W3, b3):
    G, R = _G, _REC
    uid = x[:, 0]
    u1c = x[:, 1:1 + G]
    u2c = x[:, 1 + G:1 + 2 * G]
    ur = x[:, 1 + 2 * G:1 + 2 * G + R]
    ic = 1 + 2 * G + R
    iid = x[:, ic]
    i1c = x[:, ic + 1:ic + 1 + G]
    i2c = x[:, ic + 1 + G:ic + 1 + 2 * G]
    ir = x[:, ic + 1 + 2 * G:]
    ur64 = jnp.pad(ur, ((0, 0), (0, _K - R))).astype(jnp.int32)
    ir64 = jnp.pad(ir, ((0, 0), (0, _K - R))).astype(jnp.int32)
    bf16 = jnp.bfloat16

    ue, ie, urs, irs = _sc_gather(
        uid.astype(jnp.int32), iid.astype(jnp.int32),
        ur64.reshape(-1), ir64.reshape(-1),
        user_embed.astype(bf16), item_embed.astype(bf16))
    perm = (list(range(0, 32, 2)) + list(range(1, 32, 2))
            + list(range(32, 64, 2)) + list(range(33, 64, 2)))
    perm = jnp.asarray(perm, jnp.int32)
    irow0 = item_embed[0:1].astype(bf16).astype(jnp.float32)[:, perm]
    urow0 = user_embed[0:1].astype(bf16).astype(jnp.float32)[:, perm]
    return _tc_dense(
        u1c, u2c, i1c, i2c, ue, ie, urs, irs,
        ur64, ir64, irow0, urow0,
        u_macro_embed, i_macro_embed, uV, iV,
        jnp.take(uV, perm, axis=0), jnp.take(iV, perm, axis=0),
        W1, b1.reshape(1, -1), alpha1.reshape(1, 1),
        W2, b2.reshape(1, -1), alpha2.reshape(1, 1),
        W3.reshape(1, -1), b3.reshape(1, 1))
